# baseline (speedup 1.0000x reference)
import functools

import jax
import jax.numpy as jnp
from jax import lax
from jax.experimental import pallas as pl
from jax.experimental.pallas import tpu as pltpu
from jax.experimental.pallas import tpu_sc as plsc

N_GENES = 100000
EMB_DIM = 64
BATCH = 16384

_NC = 2
_NS = 16
_NW = _NC * _NS
_B_PER_W = BATCH // _NW

_mesh = plsc.VectorSubcoreMesh(core_axis_name="c", subcore_axis_name="s")


@functools.partial(
    pl.kernel,
    mesh=_mesh,
    compiler_params=pltpu.CompilerParams(use_tc_tiling_on_sc=False),
    out_type=(
        jax.ShapeDtypeStruct((BATCH, EMB_DIM), jnp.float32),
        jax.ShapeDtypeStruct((BATCH, EMB_DIM), jnp.float32),
    ),
    scratch_types=[
        pltpu.VMEM((_B_PER_W,), jnp.int32),
        pltpu.VMEM((_B_PER_W, EMB_DIM), jnp.float32),
        pltpu.VMEM((_B_PER_W, EMB_DIM), jnp.float32),
        pltpu.SemaphoreType.DMA,
        pltpu.SemaphoreType.DMA,
    ],
)
def _gene_embed(idx_hbm, mu_hbm, ls_hbm, mu_out, ls_out,
                idx_v, mu_v, ls_v, sem_mu, sem_ls):
    wid = lax.axis_index("s") * _NC + lax.axis_index("c")
    base = wid * _B_PER_W
    pltpu.sync_copy(idx_hbm.at[pl.ds(base, _B_PER_W)], idx_v)
    c_mu = pltpu.async_copy(mu_hbm.at[idx_v], mu_v, sem_mu)
    c_ls = pltpu.async_copy(ls_hbm.at[idx_v], ls_v, sem_ls)
    c_mu.wait()
    pltpu.sync_copy(mu_v, mu_out.at[pl.ds(base, _B_PER_W)])
    c_ls.wait()
    pltpu.sync_copy(ls_v, ls_out.at[pl.ds(base, _B_PER_W)])


def kernel(indices, emb_mu_w, emb_log_sigma_w):
    idx = indices.astype(jnp.int32)
    mu, log_sigma = _gene_embed(idx, emb_mu_w, emb_log_sigma_w)
    return (mu, log_sigma)
